# Initial kernel scaffold; baseline (speedup 1.0000x reference)
import jax
import jax.numpy as jnp
from jax.experimental import pallas as pl


def kernel(x, edge_index, params, Wc, bc):
    raise NotImplementedError("write your pallas kernel here")


# speedup vs baseline: 9.4135x; 9.4135x over previous
import functools

import jax
import jax.numpy as jnp
from jax import lax
from jax.experimental import pallas as pl
from jax.experimental.pallas import tpu as pltpu
from jax.experimental.pallas import tpu_sc as plsc

N = 10000
NPAD = 10240
E = 320000
D_IN = 128
HID = 256
D_OUT = 64
KSEL = 256

NSUB = 16
RPS = NPAD // NSUB

_mesh = plsc.VectorSubcoreMesh(core_axis_name="c", subcore_axis_name="s",
                               num_cores=2, num_subcores=NSUB)


_DCH = 2000
_DNC = (E // 2) // _DCH


@functools.partial(
    pl.kernel,
    out_type=jax.ShapeDtypeStruct((2, NSUB, RPS * 16), jnp.float32),
    mesh=_mesh,
    compiler_params=pltpu.CompilerParams(needs_layout_passes=False),
    scratch_types=[
        pltpu.VMEM((_DCH,), jnp.int32),
        pltpu.VMEM((RPS * 16,), jnp.float32),
    ],
)
def _sc_degree(dst_hbm, out_hbm, dbuf, acc):
    c = lax.axis_index("c")
    s = lax.axis_index("s")
    base = s * RPS
    zero16 = jnp.zeros((16,), jnp.float32)
    one16 = jnp.ones((16,), jnp.float32)
    lane = lax.iota(jnp.int32, 16)

    def z(i, _):
        acc[pl.ds(i * 16, 16)] = zero16
        return 0

    lax.fori_loop(0, RPS, z, 0)

    def chunk(t, _):
        pltpu.sync_copy(dst_hbm.at[c, t], dbuf)

        def b(i, _):
            idx = dbuf[pl.ds(i * 16, 16)]
            mask = (idx >= base) & (idx < base + RPS)
            addr = jnp.where(mask, (idx - base) * 16 + lane, lane)
            val = jnp.where(mask, one16, 0.0)
            plsc.addupdate_scatter(acc, [addr], val)
            return 0

        lax.fori_loop(0, _DCH // 16, b, 0)
        return 0

    lax.fori_loop(0, _DNC, chunk, 0)
    pltpu.sync_copy(acc, out_hbm.at[c, s])


_B = 80
_G = 25
_CH1 = 125


@functools.partial(
    pl.kernel,
    out_type=jax.ShapeDtypeStruct((2, NPAD, D_IN), jnp.float32),
    mesh=_mesh,
    scratch_types=[
        pltpu.VMEM((_G, _B), jnp.int32),
        pltpu.VMEM((_G, _B), jnp.int32),
        pltpu.VMEM((_B, D_IN), jnp.float32),
        pltpu.VMEM_SHARED((NPAD, D_IN), jnp.float32),
        pltpu.SemaphoreType.DMA,
    ],
)
def _sc_scatter1(g_hbm, src_hbm, dst_hbm, out_hbm, sbuf, dbuf, rows, acc, sem):
    c = lax.axis_index("c")
    s = lax.axis_index("s")
    zero16 = jnp.zeros((16,), jnp.float32)

    def z(i, _):
        def zi(k, _):
            rows[i, pl.ds(k * 16, 16)] = zero16
            return 0
        lax.fori_loop(0, D_IN // 16, zi, 0)
        return 0

    lax.fori_loop(0, _B, z, 0)

    def zc(k, _):
        pltpu.sync_copy(rows, acc.at[pl.ds(s * RPS + k * _B, _B)])
        return 0

    lax.fori_loop(0, RPS // _B, zc, 0)
    plsc.subcore_barrier()

    def group(gi, _):
        pltpu.sync_copy(src_hbm.at[c, s, gi], sbuf)
        pltpu.sync_copy(dst_hbm.at[c, s, gi], dbuf)

        def chunk(j, _):
            pltpu.async_copy(g_hbm.at[sbuf.at[j]], rows, sem).wait()
            pltpu.sync_copy(rows, acc.at[dbuf.at[j]], add=True)
            return 0

        lax.fori_loop(0, _G, chunk, 0)
        return 0

    lax.fori_loop(0, _CH1 // _G, group, 0)
    plsc.subcore_barrier()
    pltpu.sync_copy(acc.at[pl.ds(s * RPS, RPS)],
                    out_hbm.at[c, pl.ds(s * RPS, RPS)])


_CH2 = 250


@functools.partial(
    pl.kernel,
    out_type=jax.ShapeDtypeStruct((2, NPAD, 128), jnp.float32),
    mesh=_mesh,
    scratch_types=[
        pltpu.VMEM((_G, _B), jnp.int32),
        pltpu.VMEM((_G, _B), jnp.int32),
        pltpu.VMEM((_B, 128), jnp.float32),
        pltpu.VMEM_SHARED((NPAD, 128), jnp.float32),
        pltpu.SemaphoreType.DMA,
    ],
)
def _sc_scatter2(g2_hbm, src_hbm, dst_hbm, out_hbm, sbuf, dbuf, rows, acc, sem):
    c = lax.axis_index("c")
    s = lax.axis_index("s")
    zero16 = jnp.zeros((16,), jnp.float32)

    def z(i, _):
        def zi(k, _):
            rows[i, pl.ds(k * 16, 16)] = zero16
            return 0
        lax.fori_loop(0, 128 // 16, zi, 0)
        return 0

    lax.fori_loop(0, _B, z, 0)

    def zc(k, _):
        pltpu.sync_copy(rows, acc.at[pl.ds(s * RPS + k * _B, _B)])
        return 0

    lax.fori_loop(0, RPS // _B, zc, 0)
    plsc.subcore_barrier()
    off = c * NPAD

    def group(gi, _):
        pltpu.sync_copy(src_hbm.at[s, gi], sbuf)
        pltpu.sync_copy(dst_hbm.at[s, gi], dbuf)

        def badd(i, _):
            def bi(k, _):
                sbuf[i, pl.ds(k * 16, 16)] = sbuf[i, pl.ds(k * 16, 16)] + off
                return 0
            lax.fori_loop(0, _B // 16, bi, 0)
            return 0

        lax.fori_loop(0, _G, badd, 0)

        def chunk(j, _):
            pltpu.async_copy(g2_hbm.at[sbuf.at[j]], rows, sem).wait()
            pltpu.sync_copy(rows, acc.at[dbuf.at[j]], add=True)
            return 0

        lax.fori_loop(0, _G, chunk, 0)
        return 0

    lax.fori_loop(0, _CH2 // _G, group, 0)
    plsc.subcore_barrier()
    pltpu.sync_copy(acc.at[pl.ds(s * RPS, RPS)],
                    out_hbm.at[c, pl.ds(s * RPS, RPS)])


_R = 1024


def _deg_col(deg_ref):
    parts = deg_ref[...]
    deg = jnp.sum(parts, axis=1, keepdims=True)
    return lax.rsqrt(deg + 1.0)


def _prep_body(deg_ref, x_ref, g_ref):
    g_ref[...] = x_ref[...] * _deg_col(deg_ref)


def _prep(deg32, xpad):
    return pl.pallas_call(
        _prep_body,
        grid=(NPAD // _R,),
        in_specs=[pl.BlockSpec((_R, 32), lambda i: (i, 0)),
                  pl.BlockSpec((_R, D_IN), lambda i: (i, 0))],
        out_specs=pl.BlockSpec((_R, D_IN), lambda i: (i, 0)),
        out_shape=jax.ShapeDtypeStruct((NPAD, D_IN), jnp.float32),
    )(deg32, xpad)


def _topk_body(h_ref, p_ref, vals_ref, rows_ref, *, k):
    p2 = p_ref[...]
    nrm = jnp.sqrt(jnp.sum(p2 * p2))
    inv = 1.0 / (nrm + 1e-12)
    ch = NPAD // 8
    ys = []
    for r in range(8):
        hr = h_ref[pl.ds(r * ch, ch), :]
        ys.append(lax.dot_general(p2, hr, (((1,), (1,)), ((), ())),
                                  preferred_element_type=jnp.float32))
    y8 = jnp.concatenate(ys, axis=0) * inv
    ids = (lax.broadcasted_iota(jnp.int32, (8, ch), 0) * ch
           + lax.broadcasted_iota(jnp.int32, (8, ch), 1))
    neg = jnp.float32(-3.0e38)
    y8 = jnp.where(ids < N, y8, neg)

    def body(step, y):
        m = jnp.max(y)
        pick = jnp.min(jnp.where(y == m, ids, jnp.int32(2**30)))
        vals_ref[pl.ds(step, 1), :] = jnp.reshape(m, (1, 1))
        rows_ref[pl.ds(step, 1), :] = h_ref[pl.ds(pick, 1), :]
        return jnp.where(ids == pick, neg, y)

    lax.fori_loop(0, KSEL, body, y8)


def _topk(hpad, p2, k):
    return pl.pallas_call(
        functools.partial(_topk_body, k=k),
        out_shape=(jax.ShapeDtypeStruct((KSEL, 1), jnp.float32),
                   jax.ShapeDtypeStruct((KSEL, k), jnp.float32)),
    )(hpad, p2)


def _gru_body(rows_ref, vals_ref, h_ref, wz_ref, uz_ref, wr_ref, ur_ref,
              wh_ref, uh_ref, bz_ref, br_ref, bh_ref, hn_ref):
    xt = rows_ref[...] * jnp.tanh(vals_ref[...])
    h = h_ref[...]
    f32 = jnp.float32
    zg = jax.nn.sigmoid(jnp.dot(xt, wz_ref[...], preferred_element_type=f32)
                        + jnp.dot(h, uz_ref[...], preferred_element_type=f32)
                        + bz_ref[...])
    rg = jax.nn.sigmoid(jnp.dot(xt, wr_ref[...], preferred_element_type=f32)
                        + jnp.dot(h, ur_ref[...], preferred_element_type=f32)
                        + br_ref[...])
    ht = jnp.tanh(jnp.dot(xt, wh_ref[...], preferred_element_type=f32)
                  + jnp.dot(rg * h, uh_ref[...], preferred_element_type=f32)
                  + bh_ref[...])
    hn_ref[...] = (1.0 - zg) * h + zg * ht


def _gru(rows, vals, ht_w, lp):
    ic = ht_w.shape[1]
    return pl.pallas_call(
        _gru_body,
        out_shape=jax.ShapeDtypeStruct((KSEL, ic), jnp.float32),
    )(rows, vals, ht_w, lp["Wz"], lp["Uz"], lp["Wr"], lp["Ur"],
      lp["Wh"], lp["Uh"], lp["bz"].reshape(1, ic), lp["br"].reshape(1, ic),
      lp["bh"].reshape(1, ic))


def _layer1_body(deg_ref, g_ref, a1_ref, hn_ref, h_ref, gh_ref):
    dis = _deg_col(deg_ref)
    u = (a1_ref[0] + a1_ref[1] + g_ref[...]) * dis
    acc = lax.dot_general(u, hn_ref[...], (((1,), (1,)), ((), ())),
                          preferred_element_type=jnp.float32)
    h = jnp.maximum(acc, 0.0)
    h_ref[...] = h
    gh_ref[0] = h[:, :128] * dis
    gh_ref[1] = h[:, 128:] * dis


def _layer1(deg32, g, a1, hn1):
    return pl.pallas_call(
        _layer1_body,
        grid=(NPAD // _R,),
        in_specs=[pl.BlockSpec((_R, 32), lambda i: (i, 0)),
                  pl.BlockSpec((_R, D_IN), lambda i: (i, 0)),
                  pl.BlockSpec((2, _R, D_IN), lambda i: (0, i, 0)),
                  pl.BlockSpec((HID, D_IN), lambda i: (0, 0))],
        out_specs=[pl.BlockSpec((_R, HID), lambda i: (i, 0)),
                   pl.BlockSpec((2, _R, 128), lambda i: (0, i, 0))],
        out_shape=(jax.ShapeDtypeStruct((NPAD, HID), jnp.float32),
                   jax.ShapeDtypeStruct((2, NPAD, 128), jnp.float32)),
    )(deg32, g, a1, hn1)


def _layer2_body(deg_ref, gh_ref, a2_ref, hn_ref, wc_ref, bc_ref, out_ref):
    dis = _deg_col(deg_ref)
    u = jnp.concatenate(
        [(a2_ref[0] + gh_ref[0]) * dis,
         (a2_ref[1] + gh_ref[1]) * dis], axis=1)
    acc = lax.dot_general(u, hn_ref[...], (((1,), (1,)), ((), ())),
                          preferred_element_type=jnp.float32)
    h2 = jnp.maximum(acc, 0.0)
    out_ref[...] = (jnp.dot(h2, wc_ref[...], preferred_element_type=jnp.float32)
                    + bc_ref[...])


def _layer2(deg32, gh, a2, hn2, wc, bc2):
    return pl.pallas_call(
        _layer2_body,
        grid=(NPAD // _R,),
        in_specs=[pl.BlockSpec((_R, 32), lambda i: (i, 0)),
                  pl.BlockSpec((2, _R, 128), lambda i: (0, i, 0)),
                  pl.BlockSpec((2, _R, 128), lambda i: (0, i, 0)),
                  pl.BlockSpec((HID, HID), lambda i: (0, 0)),
                  pl.BlockSpec((HID, D_OUT), lambda i: (0, 0)),
                  pl.BlockSpec((1, D_OUT), lambda i: (0, 0))],
        out_specs=pl.BlockSpec((_R, D_OUT), lambda i: (i, 0)),
        out_shape=jax.ShapeDtypeStruct((N, D_OUT), jnp.float32),
    )(deg32, gh, a2, hn2, wc, bc2)


def kernel(x, edge_index, params, Wc, bc):
    src = edge_index[0]
    dst = edge_index[1]
    src1 = src.reshape(2, NSUB, _CH1 // _G, _G, _B)
    dst1 = dst.reshape(2, NSUB, _CH1 // _G, _G, _B)
    src2 = src.reshape(NSUB, _CH2 // _G, _G, _B)
    dst2 = dst.reshape(NSUB, _CH2 // _G, _G, _B)
    dstd = dst.reshape(2, _DNC, _DCH)
    xpad = jnp.pad(x, ((0, NPAD - N), (0, 0)))

    degp = _sc_degree(dstd)
    deg32 = (degp.reshape(2, NPAD, 16)
             .transpose(1, 0, 2).reshape(NPAD, 32))
    g = _prep(deg32, xpad)

    vals1, rows1 = _topk(xpad, params[0]["p"].reshape(1, D_IN), D_IN)
    hn1 = _gru(rows1, vals1, params[0]["W"].T, params[0])

    a1 = _sc_scatter1(g, src1, dst1)
    h1, gh = _layer1(deg32, g, a1, hn1)

    vals2, rows2 = _topk(h1, params[1]["p"].reshape(1, HID), HID)
    hn2 = _gru(rows2, vals2, params[1]["W"].T, params[1])

    a2 = _sc_scatter2(gh.reshape(2 * NPAD, 128), src2, dst2)
    out = _layer2(deg32, gh, a2, hn2, Wc, bc.reshape(1, D_OUT))

    return (out, hn1.T, hn2.T)


# speedup vs baseline: 11.5663x; 1.2287x over previous
import functools

import jax
import jax.numpy as jnp
from jax import lax
from jax.experimental import pallas as pl
from jax.experimental.pallas import tpu as pltpu
from jax.experimental.pallas import tpu_sc as plsc

N = 10000
NPAD = 10240
E = 320000
D_IN = 128
HID = 256
D_OUT = 64
KSEL = 256

NSUB = 16
RPS = NPAD // NSUB

_mesh = plsc.VectorSubcoreMesh(core_axis_name="c", subcore_axis_name="s",
                               num_cores=2, num_subcores=NSUB)


_DCH = 2000
_DNC = (E // NSUB) // _DCH
_DR = NPAD // 2


@functools.partial(
    pl.kernel,
    out_type=jax.ShapeDtypeStruct((2, NSUB, _DR * 16), jnp.float32),
    mesh=_mesh,
    compiler_params=pltpu.CompilerParams(needs_layout_passes=False),
    scratch_types=[
        pltpu.VMEM((_DCH,), jnp.int32),
        pltpu.VMEM((_DR * 16,), jnp.float32),
    ],
)
def _sc_degree(dst_hbm, out_hbm, dbuf, acc):
    c = lax.axis_index("c")
    s = lax.axis_index("s")
    base = c * _DR
    zero16 = jnp.zeros((16,), jnp.float32)
    one16 = jnp.ones((16,), jnp.float32)
    lane = lax.iota(jnp.int32, 16)

    def z(i, _):
        acc[pl.ds(i * 16, 16)] = zero16
        return 0

    lax.fori_loop(0, _DR, z, 0, unroll=8)

    def chunk(t, _):
        pltpu.sync_copy(dst_hbm.at[s, t], dbuf)

        def b(i, _):
            idx = dbuf[pl.ds(i * 16, 16)]
            mask = (idx >= base) & (idx < base + _DR)
            addr = jnp.where(mask, (idx - base) * 16 + lane, lane)
            val = jnp.where(mask, one16, 0.0)
            plsc.addupdate_scatter(acc, [addr], val)
            return 0

        lax.fori_loop(0, _DCH // 16, b, 0, unroll=4)
        return 0

    lax.fori_loop(0, _DNC, chunk, 0)
    pltpu.sync_copy(acc, out_hbm.at[c, s])


_B = 80
_G = 5
_NG1 = 25
_NG2 = 50


def _zero_rows(rows, ncol):
    zero16 = jnp.zeros((16,), jnp.float32)

    def z(i, _):
        def zi(k, _):
            rows[i, pl.ds(k * 16, 16)] = zero16
            return 0
        lax.fori_loop(0, ncol // 16, zi, 0)
        return 0

    lax.fori_loop(0, _B, z, 0, unroll=2)


def _zero_acc(rows, acc, s):
    def zc(k, _):
        pltpu.sync_copy(rows.at[pl.ds(0, 80)],
                        acc.at[pl.ds(s * RPS + k * 80, 80)])
        return 0

    lax.fori_loop(0, RPS // 80, zc, 0)


def _pipelined_groups(g_hbm, idx_view, dst_view, ng, sbuf, dbuf,
                      rows0, rows1, acc, sem0, sem1):
    bufs = (rows0, rows1)
    sems = (sem0, sem1)

    def group(gi, _):
        pltpu.sync_copy(idx_view(gi), sbuf)
        pltpu.sync_copy(dst_view(gi), dbuf)
        d = pltpu.async_copy(g_hbm.at[sbuf.at[0]], rows0, sem0)
        for j in range(_G):
            dn = None
            if j + 1 < _G:
                dn = pltpu.async_copy(g_hbm.at[sbuf.at[j + 1]],
                                      bufs[(j + 1) % 2], sems[(j + 1) % 2])
            d.wait()
            pltpu.sync_copy(bufs[j % 2], acc.at[dbuf.at[j]], add=True)
            d = dn
        return 0

    lax.fori_loop(0, ng, group, 0)


@functools.partial(
    pl.kernel,
    out_type=jax.ShapeDtypeStruct((2, NPAD, D_IN), jnp.float32),
    mesh=_mesh,
    scratch_types=[
        pltpu.VMEM((_G, _B), jnp.int32),
        pltpu.VMEM((_G, _B), jnp.int32),
        pltpu.VMEM((_B, D_IN), jnp.float32),
        pltpu.VMEM((_B, D_IN), jnp.float32),
        pltpu.VMEM_SHARED((NPAD, D_IN), jnp.float32),
        pltpu.SemaphoreType.DMA,
        pltpu.SemaphoreType.DMA,
    ],
)
def _sc_scatter1(g_hbm, src_hbm, dst_hbm, out_hbm, sbuf, dbuf,
                 rows0, rows1, acc, sem0, sem1):
    c = lax.axis_index("c")
    s = lax.axis_index("s")
    _zero_rows(rows0, D_IN)
    _zero_acc(rows0, acc, s)
    plsc.subcore_barrier()
    _pipelined_groups(g_hbm,
                      lambda gi: src_hbm.at[c, s, gi],
                      lambda gi: dst_hbm.at[c, s, gi],
                      _NG1, sbuf, dbuf, rows0, rows1, acc, sem0, sem1)
    plsc.subcore_barrier()
    pltpu.sync_copy(acc.at[pl.ds(s * RPS, RPS)],
                    out_hbm.at[c, pl.ds(s * RPS, RPS)])


@functools.partial(
    pl.kernel,
    out_type=jax.ShapeDtypeStruct((2, NPAD, 128), jnp.float32),
    mesh=_mesh,
    scratch_types=[
        pltpu.VMEM((_G, _B), jnp.int32),
        pltpu.VMEM((_G, _B), jnp.int32),
        pltpu.VMEM((_B, 128), jnp.float32),
        pltpu.VMEM((_B, 128), jnp.float32),
        pltpu.VMEM_SHARED((NPAD, 128), jnp.float32),
        pltpu.SemaphoreType.DMA,
        pltpu.SemaphoreType.DMA,
    ],
)
def _sc_scatter2(g2_hbm, src_hbm, dst_hbm, out_hbm, sbuf, dbuf,
                 rows0, rows1, acc, sem0, sem1):
    c = lax.axis_index("c")
    s = lax.axis_index("s")
    _zero_rows(rows0, 128)
    _zero_acc(rows0, acc, s)
    plsc.subcore_barrier()
    _pipelined_groups(g2_hbm,
                      lambda gi: src_hbm.at[c, s, gi],
                      lambda gi: dst_hbm.at[s, gi],
                      _NG2, sbuf, dbuf, rows0, rows1, acc, sem0, sem1)
    plsc.subcore_barrier()
    pltpu.sync_copy(acc.at[pl.ds(s * RPS, RPS)],
                    out_hbm.at[c, pl.ds(s * RPS, RPS)])


_R = 1024


def _deg_col(deg_ref):
    parts = deg_ref[...]
    deg = jnp.sum(parts, axis=1, keepdims=True)
    return lax.rsqrt(deg + 1.0)


def _prep_body(deg_ref, x_ref, g_ref):
    g_ref[...] = x_ref[...] * _deg_col(deg_ref)


def _prep(deg32, xpad):
    return pl.pallas_call(
        _prep_body,
        grid=(NPAD // _R,),
        in_specs=[pl.BlockSpec((_R, 256), lambda i: (i, 0)),
                  pl.BlockSpec((_R, D_IN), lambda i: (i, 0))],
        out_specs=pl.BlockSpec((_R, D_IN), lambda i: (i, 0)),
        out_shape=jax.ShapeDtypeStruct((NPAD, D_IN), jnp.float32),
    )(deg32, xpad)


def _topk_body(h_ref, p_ref, vals_ref, rows_ref, *, k):
    p2 = p_ref[...]
    nrm = jnp.sqrt(jnp.sum(p2 * p2))
    inv = 1.0 / (nrm + 1e-12)
    ch = NPAD // 8
    ys = []
    for r in range(8):
        hr = h_ref[pl.ds(r * ch, ch), :]
        ys.append(lax.dot_general(p2, hr, (((1,), (1,)), ((), ())),
                                  preferred_element_type=jnp.float32))
    y8 = jnp.concatenate(ys, axis=0) * inv
    ids = (lax.broadcasted_iota(jnp.int32, (8, ch), 0) * ch
           + lax.broadcasted_iota(jnp.int32, (8, ch), 1))
    neg = jnp.float32(-3.0e38)
    y8 = jnp.where(ids < N, y8, neg)

    def body(step, y):
        m = jnp.max(y)
        pick = jnp.min(jnp.where(y == m, ids, jnp.int32(2**30)))
        vals_ref[pl.ds(step, 1), :] = jnp.reshape(m, (1, 1))
        rows_ref[pl.ds(step, 1), :] = h_ref[pl.ds(pick, 1), :]
        return jnp.where(ids == pick, neg, y)

    lax.fori_loop(0, KSEL, body, y8)


def _topk(hpad, p2, k):
    return pl.pallas_call(
        functools.partial(_topk_body, k=k),
        out_shape=(jax.ShapeDtypeStruct((KSEL, 1), jnp.float32),
                   jax.ShapeDtypeStruct((KSEL, k), jnp.float32)),
    )(hpad, p2)


def _gru_body(rows_ref, vals_ref, h_ref, wz_ref, uz_ref, wr_ref, ur_ref,
              wh_ref, uh_ref, bz_ref, br_ref, bh_ref, hn_ref):
    xt = rows_ref[...] * jnp.tanh(vals_ref[...])
    h = h_ref[...]
    f32 = jnp.float32
    zg = jax.nn.sigmoid(jnp.dot(xt, wz_ref[...], preferred_element_type=f32)
                        + jnp.dot(h, uz_ref[...], preferred_element_type=f32)
                        + bz_ref[...])
    rg = jax.nn.sigmoid(jnp.dot(xt, wr_ref[...], preferred_element_type=f32)
                        + jnp.dot(h, ur_ref[...], preferred_element_type=f32)
                        + br_ref[...])
    ht = jnp.tanh(jnp.dot(xt, wh_ref[...], preferred_element_type=f32)
                  + jnp.dot(rg * h, uh_ref[...], preferred_element_type=f32)
                  + bh_ref[...])
    hn_ref[...] = (1.0 - zg) * h + zg * ht


def _gru(rows, vals, ht_w, lp):
    ic = ht_w.shape[1]
    return pl.pallas_call(
        _gru_body,
        out_shape=jax.ShapeDtypeStruct((KSEL, ic), jnp.float32),
    )(rows, vals, ht_w, lp["Wz"], lp["Uz"], lp["Wr"], lp["Ur"],
      lp["Wh"], lp["Uh"], lp["bz"].reshape(1, ic), lp["br"].reshape(1, ic),
      lp["bh"].reshape(1, ic))


def _layer1_body(deg_ref, g_ref, a1_ref, hn_ref, h_ref, gh_ref):
    dis = _deg_col(deg_ref)
    u = (a1_ref[0] + a1_ref[1] + g_ref[...]) * dis
    acc = lax.dot_general(u, hn_ref[...], (((1,), (1,)), ((), ())),
                          preferred_element_type=jnp.float32)
    h = jnp.maximum(acc, 0.0)
    h_ref[...] = h
    gh_ref[0] = h[:, :128] * dis
    gh_ref[1] = h[:, 128:] * dis


def _layer1(deg32, g, a1, hn1):
    return pl.pallas_call(
        _layer1_body,
        grid=(NPAD // _R,),
        in_specs=[pl.BlockSpec((_R, 256), lambda i: (i, 0)),
                  pl.BlockSpec((_R, D_IN), lambda i: (i, 0)),
                  pl.BlockSpec((2, _R, D_IN), lambda i: (0, i, 0)),
                  pl.BlockSpec((HID, D_IN), lambda i: (0, 0))],
        out_specs=[pl.BlockSpec((_R, HID), lambda i: (i, 0)),
                   pl.BlockSpec((2, _R, 128), lambda i: (0, i, 0))],
        out_shape=(jax.ShapeDtypeStruct((NPAD, HID), jnp.float32),
                   jax.ShapeDtypeStruct((2, NPAD, 128), jnp.float32)),
    )(deg32, g, a1, hn1)


def _layer2_body(deg_ref, gh_ref, a2_ref, hn_ref, wc_ref, bc_ref, out_ref):
    dis = _deg_col(deg_ref)
    u = jnp.concatenate(
        [(a2_ref[0] + gh_ref[0]) * dis,
         (a2_ref[1] + gh_ref[1]) * dis], axis=1)
    acc = lax.dot_general(u, hn_ref[...], (((1,), (1,)), ((), ())),
                          preferred_element_type=jnp.float32)
    h2 = jnp.maximum(acc, 0.0)
    out_ref[...] = (jnp.dot(h2, wc_ref[...], preferred_element_type=jnp.float32)
                    + bc_ref[...])


def _layer2(deg32, gh, a2, hn2, wc, bc2):
    return pl.pallas_call(
        _layer2_body,
        grid=(NPAD // _R,),
        in_specs=[pl.BlockSpec((_R, 256), lambda i: (i, 0)),
                  pl.BlockSpec((2, _R, 128), lambda i: (0, i, 0)),
                  pl.BlockSpec((2, _R, 128), lambda i: (0, i, 0)),
                  pl.BlockSpec((HID, HID), lambda i: (0, 0)),
                  pl.BlockSpec((HID, D_OUT), lambda i: (0, 0)),
                  pl.BlockSpec((1, D_OUT), lambda i: (0, 0))],
        out_specs=pl.BlockSpec((_R, D_OUT), lambda i: (i, 0)),
        out_shape=jax.ShapeDtypeStruct((N, D_OUT), jnp.float32),
    )(deg32, gh, a2, hn2, wc, bc2)


def kernel(x, edge_index, params, Wc, bc):
    src = edge_index[0]
    dst = edge_index[1]
    src1 = src.reshape(2, NSUB, _NG1, _G, _B)
    dst1 = dst.reshape(2, NSUB, _NG1, _G, _B)
    src2 = src.reshape(NSUB, _NG2, _G, _B)
    srcb2 = jnp.stack([src2, src2 + NPAD])
    dst2 = dst.reshape(NSUB, _NG2, _G, _B)
    dstd = dst.reshape(NSUB, _DNC, _DCH)
    xpad = jnp.pad(x, ((0, NPAD - N), (0, 0)))

    degp = _sc_degree(dstd)
    deg32 = (degp.reshape(2, NSUB, _DR, 16)
             .transpose(0, 2, 1, 3).reshape(NPAD, NSUB * 16))
    g = _prep(deg32, xpad)

    vals1, rows1 = _topk(xpad, params[0]["p"].reshape(1, D_IN), D_IN)
    hn1 = _gru(rows1, vals1, params[0]["W"].T, params[0])

    a1 = _sc_scatter1(g, src1, dst1)
    h1, gh = _layer1(deg32, g, a1, hn1)

    vals2, rows2 = _topk(h1, params[1]["p"].reshape(1, HID), HID)
    hn2 = _gru(rows2, vals2, params[1]["W"].T, params[1])

    a2 = _sc_scatter2(gh.reshape(2 * NPAD, 128), srcb2, dst2)
    out = _layer2(deg32, gh, a2, hn2, Wc, bc.reshape(1, D_OUT))

    return (out, hn1.T, hn2.T)


# speedup vs baseline: 14.5327x; 1.2565x over previous
import functools

import jax
import jax.numpy as jnp
from jax import lax
from jax.experimental import pallas as pl
from jax.experimental.pallas import tpu as pltpu
from jax.experimental.pallas import tpu_sc as plsc

N = 10000
NPAD = 10240
E = 320000
D_IN = 128
HID = 256
D_OUT = 64
KSEL = 256

NSUB = 16
RPS = NPAD // NSUB

_mesh = plsc.VectorSubcoreMesh(core_axis_name="c", subcore_axis_name="s",
                               num_cores=2, num_subcores=NSUB)


_DCH = 2000
_DNC = (E // NSUB) // _DCH
_DR = NPAD // 2


@functools.partial(
    pl.kernel,
    out_type=jax.ShapeDtypeStruct((2, NSUB, _DR * 16), jnp.float32),
    mesh=_mesh,
    compiler_params=pltpu.CompilerParams(needs_layout_passes=False),
    scratch_types=[
        pltpu.VMEM((_DCH,), jnp.int32),
        pltpu.VMEM((_DR * 16,), jnp.float32),
    ],
)
def _sc_degree(dst_hbm, out_hbm, dbuf, acc):
    c = lax.axis_index("c")
    s = lax.axis_index("s")
    base = c * _DR
    zero16 = jnp.zeros((16,), jnp.float32)
    one16 = jnp.ones((16,), jnp.float32)
    lane = lax.iota(jnp.int32, 16)

    def z(i, _):
        acc[pl.ds(i * 16, 16)] = zero16
        return 0

    lax.fori_loop(0, _DR, z, 0, unroll=8)

    def chunk(t, _):
        pltpu.sync_copy(dst_hbm.at[s, t], dbuf)

        def b(i, _):
            idx = dbuf[pl.ds(i * 16, 16)]
            mask = (idx >= base) & (idx < base + _DR)
            addr = jnp.where(mask, (idx - base) * 16 + lane, lane)
            val = jnp.where(mask, one16, 0.0)
            plsc.addupdate_scatter(acc, [addr], val)
            return 0

        lax.fori_loop(0, _DCH // 16, b, 0, unroll=4)
        return 0

    lax.fori_loop(0, _DNC, chunk, 0)
    pltpu.sync_copy(acc, out_hbm.at[c, s])


_B = 80
_NCH = 125


def _zero_rows(rows, ncol):
    zero16 = jnp.zeros((16,), jnp.float32)

    def z(i, _):
        def zi(k, _):
            rows[i, pl.ds(k * 16, 16)] = zero16
            return 0
        lax.fori_loop(0, ncol // 16, zi, 0)
        return 0

    lax.fori_loop(0, _B, z, 0, unroll=2)


def _zero_acc(rows, acc, s):
    def zc(k, _):
        pltpu.sync_copy(rows.at[pl.ds(0, 80)],
                        acc.at[pl.ds(s * RPS + k * 80, 80)])
        return 0

    lax.fori_loop(0, RPS // 80, zc, 0)


def _pipelined_run(g_hbm, sbuf, dbuf, nch, rows0, rows1, acc,
                   sg0, sg1, ss0, ss1, ncol):
    _zero_rows(rows1, ncol)
    pltpu.async_copy(rows1, acc.at[dbuf.at[0]], ss1, add=True)
    pltpu.async_copy(g_hbm.at[sbuf.at[0]], rows0, sg0)
    even = nch % 2 == 0

    def pair(t, _):
        e = 2 * t
        nxt = jnp.where(e + 2 < nch, e + 2, 0) if even else e + 2
        pltpu.make_async_copy(rows1, acc.at[dbuf.at[0]], ss1).wait()
        pltpu.async_copy(g_hbm.at[sbuf.at[e + 1]], rows1, sg1)
        pltpu.make_async_copy(g_hbm.at[sbuf.at[0]], rows0, sg0).wait()
        pltpu.async_copy(rows0, acc.at[dbuf.at[e]], ss0, add=True)
        pltpu.make_async_copy(rows0, acc.at[dbuf.at[0]], ss0).wait()
        pltpu.async_copy(g_hbm.at[sbuf.at[nxt]], rows0, sg0)
        pltpu.make_async_copy(g_hbm.at[sbuf.at[0]], rows1, sg1).wait()
        pltpu.async_copy(rows1, acc.at[dbuf.at[e + 1]], ss1, add=True)
        return 0

    lax.fori_loop(0, nch // 2, pair, 0)
    pltpu.make_async_copy(g_hbm.at[sbuf.at[0]], rows0, sg0).wait()
    if not even:
        pltpu.async_copy(rows0, acc.at[dbuf.at[nch - 1]], ss0, add=True)
        pltpu.make_async_copy(rows0, acc.at[dbuf.at[0]], ss0).wait()
    pltpu.make_async_copy(rows1, acc.at[dbuf.at[0]], ss1).wait()


_ST = 64


@functools.partial(
    pl.kernel,
    out_type=jax.ShapeDtypeStruct((2, NPAD, D_IN), jnp.float32),
    mesh=_mesh,
    scratch_types=[
        pltpu.VMEM((_ST, _B), jnp.int32),
        pltpu.VMEM((_ST, _B), jnp.int32),
        pltpu.VMEM((_B, D_IN), jnp.float32),
        pltpu.VMEM((_B, D_IN), jnp.float32),
        pltpu.VMEM_SHARED((NPAD, D_IN), jnp.float32),
        pltpu.SemaphoreType.DMA,
        pltpu.SemaphoreType.DMA,
        pltpu.SemaphoreType.DMA,
        pltpu.SemaphoreType.DMA,
    ],
)
def _sc_scatter1(g_hbm, src_hbm, dst_hbm, out_hbm, sbuf, dbuf,
                 rows0, rows1, acc, sg0, sg1, ss0, ss1):
    c = lax.axis_index("c")
    s = lax.axis_index("s")
    _zero_rows(rows0, D_IN)
    _zero_acc(rows0, acc, s)
    plsc.subcore_barrier()
    pltpu.sync_copy(src_hbm.at[c, s, pl.ds(0, _ST)], sbuf)
    pltpu.sync_copy(dst_hbm.at[c, s, pl.ds(0, _ST)], dbuf)
    _pipelined_run(g_hbm, sbuf, dbuf, _ST, rows0, rows1, acc,
                   sg0, sg1, ss0, ss1, D_IN)
    pltpu.sync_copy(src_hbm.at[c, s, pl.ds(_ST, _NCH - _ST)],
                    sbuf.at[pl.ds(0, _NCH - _ST)])
    pltpu.sync_copy(dst_hbm.at[c, s, pl.ds(_ST, _NCH - _ST)],
                    dbuf.at[pl.ds(0, _NCH - _ST)])
    _pipelined_run(g_hbm, sbuf, dbuf, _NCH - _ST, rows0, rows1, acc,
                   sg0, sg1, ss0, ss1, D_IN)
    plsc.subcore_barrier()
    pltpu.sync_copy(acc.at[pl.ds(s * RPS, RPS)],
                    out_hbm.at[c, pl.ds(s * RPS, RPS)])


@functools.partial(
    pl.kernel,
    out_type=jax.ShapeDtypeStruct((2, NPAD, 128), jnp.float32),
    mesh=_mesh,
    scratch_types=[
        pltpu.VMEM((_ST, _B), jnp.int32),
        pltpu.VMEM((_ST, _B), jnp.int32),
        pltpu.VMEM((_B, 128), jnp.float32),
        pltpu.VMEM((_B, 128), jnp.float32),
        pltpu.VMEM_SHARED((NPAD, 128), jnp.float32),
        pltpu.SemaphoreType.DMA,
        pltpu.SemaphoreType.DMA,
        pltpu.SemaphoreType.DMA,
        pltpu.SemaphoreType.DMA,
    ],
)
def _sc_scatter2(g2_hbm, src_hbm, dst_hbm, out_hbm, sbuf, dbuf,
                 rows0, rows1, acc, sg0, sg1, ss0, ss1):
    c = lax.axis_index("c")
    s = lax.axis_index("s")
    _zero_rows(rows0, 128)
    _zero_acc(rows0, acc, s)
    plsc.subcore_barrier()

    def run(gi, _):
        pltpu.sync_copy(src_hbm.at[c, s, gi, pl.ds(0, _ST)], sbuf)
        pltpu.sync_copy(dst_hbm.at[s, gi, pl.ds(0, _ST)], dbuf)
        _pipelined_run(g2_hbm, sbuf, dbuf, _ST, rows0, rows1, acc,
                       sg0, sg1, ss0, ss1, 128)
        pltpu.sync_copy(src_hbm.at[c, s, gi, pl.ds(_ST, _NCH - _ST)],
                        sbuf.at[pl.ds(0, _NCH - _ST)])
        pltpu.sync_copy(dst_hbm.at[s, gi, pl.ds(_ST, _NCH - _ST)],
                        dbuf.at[pl.ds(0, _NCH - _ST)])
        _pipelined_run(g2_hbm, sbuf, dbuf, _NCH - _ST, rows0, rows1, acc,
                       sg0, sg1, ss0, ss1, 128)
        return 0

    lax.fori_loop(0, 2, run, 0)
    plsc.subcore_barrier()
    pltpu.sync_copy(acc.at[pl.ds(s * RPS, RPS)],
                    out_hbm.at[c, pl.ds(s * RPS, RPS)])


_R = 1024


def _deg_col(deg_ref):
    parts = deg_ref[...]
    deg = jnp.sum(parts, axis=1, keepdims=True)
    return lax.rsqrt(deg + 1.0)


def _prep_body(deg_ref, x_ref, g_ref):
    g_ref[...] = x_ref[...] * _deg_col(deg_ref)


def _prep(deg32, xpad):
    return pl.pallas_call(
        _prep_body,
        grid=(NPAD // _R,),
        in_specs=[pl.BlockSpec((_R, 256), lambda i: (i, 0)),
                  pl.BlockSpec((_R, D_IN), lambda i: (i, 0))],
        out_specs=pl.BlockSpec((_R, D_IN), lambda i: (i, 0)),
        out_shape=jax.ShapeDtypeStruct((NPAD, D_IN), jnp.float32),
    )(deg32, xpad)


def _topk_body(h_ref, p_ref, vals_ref, rows_ref, *, k):
    p2 = p_ref[...]
    nrm = jnp.sqrt(jnp.sum(p2 * p2))
    inv = 1.0 / (nrm + 1e-12)
    ch = NPAD // 8
    ys = []
    for r in range(8):
        hr = h_ref[pl.ds(r * ch, ch), :]
        ys.append(lax.dot_general(p2, hr, (((1,), (1,)), ((), ())),
                                  preferred_element_type=jnp.float32))
    y8 = jnp.concatenate(ys, axis=0) * inv
    ids = (lax.broadcasted_iota(jnp.int32, (8, ch), 0) * ch
           + lax.broadcasted_iota(jnp.int32, (8, ch), 1))
    neg = jnp.float32(-3.0e38)
    y8 = jnp.where(ids < N, y8, neg)

    def body(step, y):
        m = jnp.max(y)
        pick = jnp.min(jnp.where(y == m, ids, jnp.int32(2**30)))
        vals_ref[pl.ds(step, 1), :] = jnp.reshape(m, (1, 1))
        rows_ref[pl.ds(step, 1), :] = h_ref[pl.ds(pick, 1), :]
        return jnp.where(ids == pick, neg, y)

    lax.fori_loop(0, KSEL, body, y8)


def _topk(hpad, p2, k):
    return pl.pallas_call(
        functools.partial(_topk_body, k=k),
        out_shape=(jax.ShapeDtypeStruct((KSEL, 1), jnp.float32),
                   jax.ShapeDtypeStruct((KSEL, k), jnp.float32)),
    )(hpad, p2)


def _gru_body(rows_ref, vals_ref, h_ref, wz_ref, uz_ref, wr_ref, ur_ref,
              wh_ref, uh_ref, bz_ref, br_ref, bh_ref, hn_ref):
    xt = rows_ref[...] * jnp.tanh(vals_ref[...])
    h = h_ref[...]
    f32 = jnp.float32
    zg = jax.nn.sigmoid(jnp.dot(xt, wz_ref[...], preferred_element_type=f32)
                        + jnp.dot(h, uz_ref[...], preferred_element_type=f32)
                        + bz_ref[...])
    rg = jax.nn.sigmoid(jnp.dot(xt, wr_ref[...], preferred_element_type=f32)
                        + jnp.dot(h, ur_ref[...], preferred_element_type=f32)
                        + br_ref[...])
    ht = jnp.tanh(jnp.dot(xt, wh_ref[...], preferred_element_type=f32)
                  + jnp.dot(rg * h, uh_ref[...], preferred_element_type=f32)
                  + bh_ref[...])
    hn_ref[...] = (1.0 - zg) * h + zg * ht


def _gru(rows, vals, ht_w, lp):
    ic = ht_w.shape[1]
    return pl.pallas_call(
        _gru_body,
        out_shape=jax.ShapeDtypeStruct((KSEL, ic), jnp.float32),
    )(rows, vals, ht_w, lp["Wz"], lp["Uz"], lp["Wr"], lp["Ur"],
      lp["Wh"], lp["Uh"], lp["bz"].reshape(1, ic), lp["br"].reshape(1, ic),
      lp["bh"].reshape(1, ic))


def _layer1_body(deg_ref, g_ref, a1_ref, hn_ref, h_ref, gh_ref):
    dis = _deg_col(deg_ref)
    u = (a1_ref[0] + a1_ref[1] + g_ref[...]) * dis
    acc = lax.dot_general(u, hn_ref[...], (((1,), (1,)), ((), ())),
                          preferred_element_type=jnp.float32)
    h = jnp.maximum(acc, 0.0)
    h_ref[...] = h
    gh_ref[0] = h[:, :128] * dis
    gh_ref[1] = h[:, 128:] * dis


def _layer1(deg32, g, a1, hn1):
    return pl.pallas_call(
        _layer1_body,
        grid=(NPAD // _R,),
        in_specs=[pl.BlockSpec((_R, 256), lambda i: (i, 0)),
                  pl.BlockSpec((_R, D_IN), lambda i: (i, 0)),
                  pl.BlockSpec((2, _R, D_IN), lambda i: (0, i, 0)),
                  pl.BlockSpec((HID, D_IN), lambda i: (0, 0))],
        out_specs=[pl.BlockSpec((_R, HID), lambda i: (i, 0)),
                   pl.BlockSpec((2, _R, 128), lambda i: (0, i, 0))],
        out_shape=(jax.ShapeDtypeStruct((NPAD, HID), jnp.float32),
                   jax.ShapeDtypeStruct((2, NPAD, 128), jnp.float32)),
    )(deg32, g, a1, hn1)


def _layer2_body(deg_ref, gh_ref, a2_ref, hn_ref, wc_ref, bc_ref, out_ref):
    dis = _deg_col(deg_ref)
    u = jnp.concatenate(
        [(a2_ref[0] + gh_ref[0]) * dis,
         (a2_ref[1] + gh_ref[1]) * dis], axis=1)
    acc = lax.dot_general(u, hn_ref[...], (((1,), (1,)), ((), ())),
                          preferred_element_type=jnp.float32)
    h2 = jnp.maximum(acc, 0.0)
    out_ref[...] = (jnp.dot(h2, wc_ref[...], preferred_element_type=jnp.float32)
                    + bc_ref[...])


def _layer2(deg32, gh, a2, hn2, wc, bc2):
    return pl.pallas_call(
        _layer2_body,
        grid=(NPAD // _R,),
        in_specs=[pl.BlockSpec((_R, 256), lambda i: (i, 0)),
                  pl.BlockSpec((2, _R, 128), lambda i: (0, i, 0)),
                  pl.BlockSpec((2, _R, 128), lambda i: (0, i, 0)),
                  pl.BlockSpec((HID, HID), lambda i: (0, 0)),
                  pl.BlockSpec((HID, D_OUT), lambda i: (0, 0)),
                  pl.BlockSpec((1, D_OUT), lambda i: (0, 0))],
        out_specs=pl.BlockSpec((_R, D_OUT), lambda i: (i, 0)),
        out_shape=jax.ShapeDtypeStruct((N, D_OUT), jnp.float32),
    )(deg32, gh, a2, hn2, wc, bc2)


def kernel(x, edge_index, params, Wc, bc):
    src = edge_index[0]
    dst = edge_index[1]
    src1 = src.reshape(2, NSUB, _NCH, _B)
    dst1 = dst.reshape(2, NSUB, _NCH, _B)
    src2 = src.reshape(NSUB, 2, _NCH, _B)
    srcb2 = jnp.stack([src2, src2 + NPAD])
    dst2 = dst.reshape(NSUB, 2, _NCH, _B)
    dstd = dst.reshape(NSUB, _DNC, _DCH)
    xpad = jnp.pad(x, ((0, NPAD - N), (0, 0)))

    degp = _sc_degree(dstd)
    deg32 = (degp.reshape(2, NSUB, _DR, 16)
             .transpose(0, 2, 1, 3).reshape(NPAD, NSUB * 16))
    g = _prep(deg32, xpad)

    vals1, rows1 = _topk(xpad, params[0]["p"].reshape(1, D_IN), D_IN)
    hn1 = _gru(rows1, vals1, params[0]["W"].T, params[0])

    a1 = _sc_scatter1(g, src1, dst1)
    h1, gh = _layer1(deg32, g, a1, hn1)

    vals2, rows2 = _topk(h1, params[1]["p"].reshape(1, HID), HID)
    hn2 = _gru(rows2, vals2, params[1]["W"].T, params[1])

    a2 = _sc_scatter2(gh.reshape(2 * NPAD, 128), srcb2, dst2)
    out = _layer2(deg32, gh, a2, hn2, Wc, bc.reshape(1, D_OUT))

    return (out, hn1.T, hn2.T)


# speedup vs baseline: 14.5437x; 1.0008x over previous
import functools

import jax
import jax.numpy as jnp
from jax import lax
from jax.experimental import pallas as pl
from jax.experimental.pallas import tpu as pltpu
from jax.experimental.pallas import tpu_sc as plsc

N = 10000
NPAD = 10240
E = 320000
D_IN = 128
HID = 256
D_OUT = 64
KSEL = 256

NSUB = 16
RPS = NPAD // NSUB

_mesh = plsc.VectorSubcoreMesh(core_axis_name="c", subcore_axis_name="s",
                               num_cores=2, num_subcores=NSUB)


_DCH = 2000
_DNC = (E // NSUB) // _DCH
_DR = NPAD // 2


@functools.partial(
    pl.kernel,
    out_type=jax.ShapeDtypeStruct((2, NSUB, _DR * 16), jnp.float32),
    mesh=_mesh,
    compiler_params=pltpu.CompilerParams(needs_layout_passes=False),
    scratch_types=[
        pltpu.VMEM((_DCH,), jnp.int32),
        pltpu.VMEM((_DR * 16,), jnp.float32),
    ],
)
def _sc_degree(dst_hbm, out_hbm, dbuf, acc):
    c = lax.axis_index("c")
    s = lax.axis_index("s")
    base = c * _DR
    zero16 = jnp.zeros((16,), jnp.float32)
    one16 = jnp.ones((16,), jnp.float32)
    lane = lax.iota(jnp.int32, 16)

    def z(i, _):
        acc[pl.ds(i * 16, 16)] = zero16
        return 0

    lax.fori_loop(0, _DR, z, 0, unroll=8)

    def chunk(t, _):
        pltpu.sync_copy(dst_hbm.at[s, t], dbuf)

        def b(i, _):
            idx = dbuf[pl.ds(i * 16, 16)]
            mask = (idx >= base) & (idx < base + _DR)
            addr = jnp.where(mask, (idx - base) * 16 + lane, lane)
            val = jnp.where(mask, one16, 0.0)
            plsc.addupdate_scatter(acc, [addr], val)
            return 0

        lax.fori_loop(0, _DCH // 16, b, 0, unroll=4)
        return 0

    lax.fori_loop(0, _DNC, chunk, 0)
    pltpu.sync_copy(acc, out_hbm.at[c, s])


_B = 80
_NCH = 125


def _zero_rows(rows, ncol):
    zero16 = jnp.zeros((16,), jnp.float32)

    def z(i, _):
        def zi(k, _):
            rows[i, pl.ds(k * 16, 16)] = zero16
            return 0
        lax.fori_loop(0, ncol // 16, zi, 0)
        return 0

    lax.fori_loop(0, _B, z, 0, unroll=2)


def _zero_acc(rows, acc, s):
    def zc(k, _):
        pltpu.sync_copy(rows.at[pl.ds(0, 80)],
                        acc.at[pl.ds(s * RPS + k * 80, 80)])
        return 0

    lax.fori_loop(0, RPS // 80, zc, 0)


def _pipelined_run(g_hbm, sbuf, dbuf, nch, rows0, rows1, acc,
                   sg0, sg1, ss0, ss1, ncol):
    _zero_rows(rows1, ncol)
    pltpu.async_copy(rows1, acc.at[dbuf.at[0]], ss1, add=True)
    pltpu.async_copy(g_hbm.at[sbuf.at[0]], rows0, sg0)
    even = nch % 2 == 0

    def pair(t, _):
        e = 2 * t
        nxt = jnp.where(e + 2 < nch, e + 2, 0) if even else e + 2
        pltpu.make_async_copy(rows1, acc.at[dbuf.at[0]], ss1).wait()
        pltpu.async_copy(g_hbm.at[sbuf.at[e + 1]], rows1, sg1)
        pltpu.make_async_copy(g_hbm.at[sbuf.at[0]], rows0, sg0).wait()
        pltpu.async_copy(rows0, acc.at[dbuf.at[e]], ss0, add=True)
        pltpu.make_async_copy(rows0, acc.at[dbuf.at[0]], ss0).wait()
        pltpu.async_copy(g_hbm.at[sbuf.at[nxt]], rows0, sg0)
        pltpu.make_async_copy(g_hbm.at[sbuf.at[0]], rows1, sg1).wait()
        pltpu.async_copy(rows1, acc.at[dbuf.at[e + 1]], ss1, add=True)
        return 0

    lax.fori_loop(0, nch // 2, pair, 0)
    pltpu.make_async_copy(g_hbm.at[sbuf.at[0]], rows0, sg0).wait()
    if not even:
        pltpu.async_copy(rows0, acc.at[dbuf.at[nch - 1]], ss0, add=True)
        pltpu.make_async_copy(rows0, acc.at[dbuf.at[0]], ss0).wait()
    pltpu.make_async_copy(rows1, acc.at[dbuf.at[0]], ss1).wait()


_ST = 64


@functools.partial(
    pl.kernel,
    out_type=jax.ShapeDtypeStruct((2, NPAD, D_IN), jnp.float32),
    mesh=_mesh,
    scratch_types=[
        pltpu.VMEM((_ST, _B), jnp.int32),
        pltpu.VMEM((_ST, _B), jnp.int32),
        pltpu.VMEM((_B, D_IN), jnp.float32),
        pltpu.VMEM((_B, D_IN), jnp.float32),
        pltpu.VMEM_SHARED((NPAD, D_IN), jnp.float32),
        pltpu.SemaphoreType.DMA,
        pltpu.SemaphoreType.DMA,
        pltpu.SemaphoreType.DMA,
        pltpu.SemaphoreType.DMA,
    ],
)
def _sc_scatter1(g_hbm, src_hbm, dst_hbm, out_hbm, sbuf, dbuf,
                 rows0, rows1, acc, sg0, sg1, ss0, ss1):
    c = lax.axis_index("c")
    s = lax.axis_index("s")
    _zero_rows(rows0, D_IN)
    _zero_acc(rows0, acc, s)
    plsc.subcore_barrier()
    pltpu.sync_copy(src_hbm.at[c, s, pl.ds(0, _ST)], sbuf)
    pltpu.sync_copy(dst_hbm.at[c, s, pl.ds(0, _ST)], dbuf)
    _pipelined_run(g_hbm, sbuf, dbuf, _ST, rows0, rows1, acc,
                   sg0, sg1, ss0, ss1, D_IN)
    pltpu.sync_copy(src_hbm.at[c, s, pl.ds(_ST, _NCH - _ST)],
                    sbuf.at[pl.ds(0, _NCH - _ST)])
    pltpu.sync_copy(dst_hbm.at[c, s, pl.ds(_ST, _NCH - _ST)],
                    dbuf.at[pl.ds(0, _NCH - _ST)])
    _pipelined_run(g_hbm, sbuf, dbuf, _NCH - _ST, rows0, rows1, acc,
                   sg0, sg1, ss0, ss1, D_IN)
    plsc.subcore_barrier()
    pltpu.sync_copy(acc.at[pl.ds(s * RPS, RPS)],
                    out_hbm.at[c, pl.ds(s * RPS, RPS)])


@functools.partial(
    pl.kernel,
    out_type=jax.ShapeDtypeStruct((2, NPAD, 128), jnp.float32),
    mesh=_mesh,
    scratch_types=[
        pltpu.VMEM((_ST, _B), jnp.int32),
        pltpu.VMEM((_ST, _B), jnp.int32),
        pltpu.VMEM((_B, 128), jnp.float32),
        pltpu.VMEM((_B, 128), jnp.float32),
        pltpu.VMEM_SHARED((NPAD, 128), jnp.float32),
        pltpu.SemaphoreType.DMA,
        pltpu.SemaphoreType.DMA,
        pltpu.SemaphoreType.DMA,
        pltpu.SemaphoreType.DMA,
    ],
)
def _sc_scatter2(g2_hbm, src_hbm, dst_hbm, out_hbm, sbuf, dbuf,
                 rows0, rows1, acc, sg0, sg1, ss0, ss1):
    c = lax.axis_index("c")
    s = lax.axis_index("s")
    _zero_rows(rows0, 128)
    _zero_acc(rows0, acc, s)
    plsc.subcore_barrier()

    off = c * NPAD

    def bias(nch, _):
        def ba(i, _):
            def bi(k, _):
                sbuf[i, pl.ds(k * 16, 16)] = sbuf[i, pl.ds(k * 16, 16)] + off
                return 0
            lax.fori_loop(0, _B // 16, bi, 0)
            return 0
        lax.fori_loop(0, nch, ba, 0)

    def run(gi, _):
        pltpu.sync_copy(src_hbm.at[s, gi, pl.ds(0, _ST)], sbuf)
        pltpu.sync_copy(dst_hbm.at[s, gi, pl.ds(0, _ST)], dbuf)
        bias(_ST, None)
        _pipelined_run(g2_hbm, sbuf, dbuf, _ST, rows0, rows1, acc,
                       sg0, sg1, ss0, ss1, 128)
        pltpu.sync_copy(src_hbm.at[s, gi, pl.ds(_ST, _NCH - _ST)],
                        sbuf.at[pl.ds(0, _NCH - _ST)])
        pltpu.sync_copy(dst_hbm.at[s, gi, pl.ds(_ST, _NCH - _ST)],
                        dbuf.at[pl.ds(0, _NCH - _ST)])
        bias(_NCH - _ST, None)
        _pipelined_run(g2_hbm, sbuf, dbuf, _NCH - _ST, rows0, rows1, acc,
                       sg0, sg1, ss0, ss1, 128)
        return 0

    lax.fori_loop(0, 2, run, 0)
    plsc.subcore_barrier()
    pltpu.sync_copy(acc.at[pl.ds(s * RPS, RPS)],
                    out_hbm.at[c, pl.ds(s * RPS, RPS)])


_R = 1024


def _deg_col(deg_ref):
    parts = deg_ref[...]
    deg = jnp.sum(parts, axis=1, keepdims=True)
    return lax.rsqrt(deg + 1.0)


def _prep_body(deg_ref, x_ref, g_ref):
    g_ref[...] = x_ref[...] * _deg_col(deg_ref)


def _prep(deg32, xpad):
    return pl.pallas_call(
        _prep_body,
        grid=(NPAD // _R,),
        in_specs=[pl.BlockSpec((_R, 256), lambda i: (i, 0)),
                  pl.BlockSpec((_R, D_IN), lambda i: (i, 0))],
        out_specs=pl.BlockSpec((_R, D_IN), lambda i: (i, 0)),
        out_shape=jax.ShapeDtypeStruct((NPAD, D_IN), jnp.float32),
    )(deg32, xpad)


def _topk_body(h_ref, p_ref, vals_ref, rows_ref, *, k):
    p2 = p_ref[...]
    nrm = jnp.sqrt(jnp.sum(p2 * p2))
    inv = 1.0 / (nrm + 1e-12)
    ch = NPAD // 8
    ys = []
    for r in range(8):
        hr = h_ref[pl.ds(r * ch, ch), :]
        ys.append(lax.dot_general(p2, hr, (((1,), (1,)), ((), ())),
                                  preferred_element_type=jnp.float32))
    y8 = jnp.concatenate(ys, axis=0) * inv
    ids = (lax.broadcasted_iota(jnp.int32, (8, ch), 0) * ch
           + lax.broadcasted_iota(jnp.int32, (8, ch), 1))
    neg = jnp.float32(-3.0e38)
    y8 = jnp.where(ids < N, y8, neg)

    def body(step, y):
        m = jnp.max(y)
        pick = jnp.min(jnp.where(y == m, ids, jnp.int32(2**30)))
        vals_ref[pl.ds(step, 1), :] = jnp.reshape(m, (1, 1))
        rows_ref[pl.ds(step, 1), :] = h_ref[pl.ds(pick, 1), :]
        return jnp.where(ids == pick, neg, y)

    lax.fori_loop(0, KSEL, body, y8)


def _topk(hpad, p2, k):
    return pl.pallas_call(
        functools.partial(_topk_body, k=k),
        out_shape=(jax.ShapeDtypeStruct((KSEL, 1), jnp.float32),
                   jax.ShapeDtypeStruct((KSEL, k), jnp.float32)),
    )(hpad, p2)


def _gru_body(rows_ref, vals_ref, h_ref, wz_ref, uz_ref, wr_ref, ur_ref,
              wh_ref, uh_ref, bz_ref, br_ref, bh_ref, hn_ref):
    xt = rows_ref[...] * jnp.tanh(vals_ref[...])
    h = h_ref[...]
    f32 = jnp.float32
    zg = jax.nn.sigmoid(jnp.dot(xt, wz_ref[...], preferred_element_type=f32)
                        + jnp.dot(h, uz_ref[...], preferred_element_type=f32)
                        + bz_ref[...])
    rg = jax.nn.sigmoid(jnp.dot(xt, wr_ref[...], preferred_element_type=f32)
                        + jnp.dot(h, ur_ref[...], preferred_element_type=f32)
                        + br_ref[...])
    ht = jnp.tanh(jnp.dot(xt, wh_ref[...], preferred_element_type=f32)
                  + jnp.dot(rg * h, uh_ref[...], preferred_element_type=f32)
                  + bh_ref[...])
    hn_ref[...] = (1.0 - zg) * h + zg * ht


def _gru(rows, vals, ht_w, lp):
    ic = ht_w.shape[1]
    return pl.pallas_call(
        _gru_body,
        out_shape=jax.ShapeDtypeStruct((KSEL, ic), jnp.float32),
    )(rows, vals, ht_w, lp["Wz"], lp["Uz"], lp["Wr"], lp["Ur"],
      lp["Wh"], lp["Uh"], lp["bz"].reshape(1, ic), lp["br"].reshape(1, ic),
      lp["bh"].reshape(1, ic))


def _layer1_body(deg_ref, g_ref, a1_ref, hn_ref, h_ref, gh_ref):
    dis = _deg_col(deg_ref)
    u = (a1_ref[0] + a1_ref[1] + g_ref[...]) * dis
    acc = lax.dot_general(u, hn_ref[...], (((1,), (1,)), ((), ())),
                          preferred_element_type=jnp.float32)
    h = jnp.maximum(acc, 0.0)
    h_ref[...] = h
    gh_ref[0] = h[:, :128] * dis
    gh_ref[1] = h[:, 128:] * dis


def _layer1(deg32, g, a1, hn1):
    return pl.pallas_call(
        _layer1_body,
        grid=(NPAD // _R,),
        in_specs=[pl.BlockSpec((_R, 256), lambda i: (i, 0)),
                  pl.BlockSpec((_R, D_IN), lambda i: (i, 0)),
                  pl.BlockSpec((2, _R, D_IN), lambda i: (0, i, 0)),
                  pl.BlockSpec((HID, D_IN), lambda i: (0, 0))],
        out_specs=[pl.BlockSpec((_R, HID), lambda i: (i, 0)),
                   pl.BlockSpec((2, _R, 128), lambda i: (0, i, 0))],
        out_shape=(jax.ShapeDtypeStruct((NPAD, HID), jnp.float32),
                   jax.ShapeDtypeStruct((2, NPAD, 128), jnp.float32)),
    )(deg32, g, a1, hn1)


def _layer2_body(deg_ref, gh_ref, a2_ref, hn_ref, wc_ref, bc_ref, out_ref):
    dis = _deg_col(deg_ref)
    u = jnp.concatenate(
        [(a2_ref[0] + gh_ref[0]) * dis,
         (a2_ref[1] + gh_ref[1]) * dis], axis=1)
    acc = lax.dot_general(u, hn_ref[...], (((1,), (1,)), ((), ())),
                          preferred_element_type=jnp.float32)
    h2 = jnp.maximum(acc, 0.0)
    out_ref[...] = (jnp.dot(h2, wc_ref[...], preferred_element_type=jnp.float32)
                    + bc_ref[...])


def _layer2(deg32, gh, a2, hn2, wc, bc2):
    return pl.pallas_call(
        _layer2_body,
        grid=(NPAD // _R,),
        in_specs=[pl.BlockSpec((_R, 256), lambda i: (i, 0)),
                  pl.BlockSpec((2, _R, 128), lambda i: (0, i, 0)),
                  pl.BlockSpec((2, _R, 128), lambda i: (0, i, 0)),
                  pl.BlockSpec((HID, HID), lambda i: (0, 0)),
                  pl.BlockSpec((HID, D_OUT), lambda i: (0, 0)),
                  pl.BlockSpec((1, D_OUT), lambda i: (0, 0))],
        out_specs=pl.BlockSpec((_R, D_OUT), lambda i: (i, 0)),
        out_shape=jax.ShapeDtypeStruct((N, D_OUT), jnp.float32),
    )(deg32, gh, a2, hn2, wc, bc2)


def kernel(x, edge_index, params, Wc, bc):
    src = edge_index[0]
    dst = edge_index[1]
    src1 = src.reshape(2, NSUB, _NCH, _B)
    dst1 = dst.reshape(2, NSUB, _NCH, _B)
    src2 = src.reshape(NSUB, 2, _NCH, _B)
    dst2 = dst.reshape(NSUB, 2, _NCH, _B)
    dstd = dst.reshape(NSUB, _DNC, _DCH)
    xpad = jnp.pad(x, ((0, NPAD - N), (0, 0)))

    degp = _sc_degree(dstd)
    deg32 = (degp.reshape(2, NSUB, _DR, 16)
             .transpose(0, 2, 1, 3).reshape(NPAD, NSUB * 16))
    g = _prep(deg32, xpad)

    vals1, rows1 = _topk(xpad, params[0]["p"].reshape(1, D_IN), D_IN)
    hn1 = _gru(rows1, vals1, params[0]["W"].T, params[0])

    a1 = _sc_scatter1(g, src1, dst1)
    h1, gh = _layer1(deg32, g, a1, hn1)

    vals2, rows2 = _topk(h1, params[1]["p"].reshape(1, HID), HID)
    hn2 = _gru(rows2, vals2, params[1]["W"].T, params[1])

    a2 = _sc_scatter2(gh.reshape(2 * NPAD, 128), src2, dst2)
    out = _layer2(deg32, gh, a2, hn2, Wc, bc.reshape(1, D_OUT))

    return (out, hn1.T, hn2.T)
